# initial kernel scaffold (unmeasured)
import jax
import jax.numpy as jnp
from jax import lax
from jax.experimental import pallas as pl
from jax.experimental.pallas import tpu as pltpu


def kernel(
    t,
):
    def body(*refs):
        pass

    out_shape = jax.ShapeDtypeStruct(..., jnp.float32)
    return pl.pallas_call(body, out_shape=out_shape)(...)


# baseline (device time: 156060 ns/iter reference)
import jax
import jax.numpy as jnp
from jax import lax
from jax.experimental import pallas as pl
from jax.experimental.pallas import tpu as pltpu

N_DEV = 4


def kernel(t):
    m_per, n = t.shape
    m_chunk = m_per // N_DEV

    def body(t_ref, out_ref, send_buf, rs_buf, ag_buf,
             rs_send_sems, rs_recv_sems, ag_send_sems, ag_recv_sems):
        my_pos = lax.axis_index("i")
        left = jnp.mod(my_pos - 1, N_DEV)
        right = jnp.mod(my_pos + 1, N_DEV)

        def chunk_rows(c):
            return pl.ds(c * m_chunk, m_chunk)

        barrier_sem = pltpu.get_barrier_semaphore()
        for nbr in [left, right]:
            pl.semaphore_signal(
                barrier_sem, inc=1,
                device_id=(nbr,), device_id_type=pl.DeviceIdType.MESH,
            )
        pl.semaphore_wait(barrier_sem, 2)

        send_buf[:, :] = t_ref[chunk_rows(my_pos), :]
        for h in range(N_DEV - 1):
            rdma = pltpu.make_async_remote_copy(
                src_ref=send_buf,
                dst_ref=rs_buf.at[h],
                send_sem=rs_send_sems.at[h],
                recv_sem=rs_recv_sems.at[h],
                device_id=(right,),
                device_id_type=pl.DeviceIdType.MESH,
            )
            rdma.start()
            rdma.wait()
            c = jnp.mod(my_pos - h - 1, N_DEV)
            if h < N_DEV - 2:
                send_buf[:, :] = rs_buf[h] + t_ref[chunk_rows(c), :]
            else:
                s = rs_buf[h] + t_ref[chunk_rows(c), :]
                r = jnp.maximum(s, 0.0)
                ag_buf[0, :, :] = jnp.tanh(s) * s * s + r * r * r

        c_own = jnp.mod(my_pos + 1, N_DEV)
        out_ref[chunk_rows(c_own), :] = ag_buf[0, :, :]
        for h in range(N_DEV - 1):
            rdma = pltpu.make_async_remote_copy(
                src_ref=ag_buf.at[h],
                dst_ref=ag_buf.at[h + 1],
                send_sem=ag_send_sems.at[h],
                recv_sem=ag_recv_sems.at[h],
                device_id=(right,),
                device_id_type=pl.DeviceIdType.MESH,
            )
            rdma.start()
            rdma.wait()
            c = jnp.mod(my_pos - h, N_DEV)
            out_ref[chunk_rows(c), :] = ag_buf[h + 1, :, :]

    return pl.pallas_call(
        body,
        out_shape=jax.ShapeDtypeStruct((m_per, n), jnp.float32),
        in_specs=[pl.BlockSpec(memory_space=pltpu.VMEM)],
        out_specs=pl.BlockSpec(memory_space=pltpu.VMEM),
        scratch_shapes=[
            pltpu.VMEM((m_chunk, n), jnp.float32),
            pltpu.VMEM((N_DEV - 1, m_chunk, n), jnp.float32),
            pltpu.VMEM((N_DEV, m_chunk, n), jnp.float32),
            pltpu.SemaphoreType.DMA((N_DEV - 1,)),
            pltpu.SemaphoreType.DMA((N_DEV - 1,)),
            pltpu.SemaphoreType.DMA((N_DEV - 1,)),
            pltpu.SemaphoreType.DMA((N_DEV - 1,)),
        ],
        compiler_params=pltpu.CompilerParams(collective_id=0),
    )(t)


# device time: 88823 ns/iter; 1.7570x vs baseline; 1.7570x over previous
import jax
import jax.numpy as jnp
from jax import lax
from jax.experimental import pallas as pl
from jax.experimental.pallas import tpu as pltpu

N_DEV = 4


def kernel(t):
    m_per, n = t.shape
    m_chunk = m_per // N_DEV
    n_half = n // 2

    def body(t_ref, out_ref,
             cw_send, cw_rs, cw_ag, ccw_send, ccw_rs, ccw_ag,
             cw_s_sems, cw_r_sems, cw_ags_sems, cw_agr_sems,
             ccw_s_sems, ccw_r_sems, ccw_ags_sems, ccw_agr_sems):
        my_pos = lax.axis_index("i")
        left = jnp.mod(my_pos - 1, N_DEV)
        right = jnp.mod(my_pos + 1, N_DEV)

        def rows(c):
            return pl.ds(c * m_chunk, m_chunk)

        cw_cols = pl.ds(0, n_half)
        ccw_cols = pl.ds(n_half, n_half)

        barrier_sem = pltpu.get_barrier_semaphore()
        for nbr in [left, right]:
            pl.semaphore_signal(
                barrier_sem, inc=1,
                device_id=(nbr,), device_id_type=pl.DeviceIdType.MESH,
            )
        pl.semaphore_wait(barrier_sem, 2)

        cw_send[:, :] = t_ref[rows(my_pos), cw_cols]
        ccw_send[:, :] = t_ref[rows(my_pos), ccw_cols]
        for h in range(N_DEV - 1):
            cw_rdma = pltpu.make_async_remote_copy(
                src_ref=cw_send, dst_ref=cw_rs.at[h],
                send_sem=cw_s_sems.at[h], recv_sem=cw_r_sems.at[h],
                device_id=(right,), device_id_type=pl.DeviceIdType.MESH,
            )
            ccw_rdma = pltpu.make_async_remote_copy(
                src_ref=ccw_send, dst_ref=ccw_rs.at[h],
                send_sem=ccw_s_sems.at[h], recv_sem=ccw_r_sems.at[h],
                device_id=(left,), device_id_type=pl.DeviceIdType.MESH,
            )
            cw_rdma.start()
            ccw_rdma.start()
            cw_rdma.wait()
            ccw_rdma.wait()
            c_cw = jnp.mod(my_pos - h - 1, N_DEV)
            c_ccw = jnp.mod(my_pos + h + 1, N_DEV)
            if h < N_DEV - 2:
                cw_send[:, :] = cw_rs[h] + t_ref[rows(c_cw), cw_cols]
                ccw_send[:, :] = ccw_rs[h] + t_ref[rows(c_ccw), ccw_cols]
            else:
                s = cw_rs[h] + t_ref[rows(c_cw), cw_cols]
                r = jnp.maximum(s, 0.0)
                cw_ag[0, :, :] = jnp.tanh(s) * s * s + r * r * r
                s = ccw_rs[h] + t_ref[rows(c_ccw), ccw_cols]
                r = jnp.maximum(s, 0.0)
                ccw_ag[0, :, :] = jnp.tanh(s) * s * s + r * r * r

        out_ref[rows(jnp.mod(my_pos + 1, N_DEV)), cw_cols] = cw_ag[0, :, :]
        out_ref[rows(jnp.mod(my_pos - 1, N_DEV)), ccw_cols] = ccw_ag[0, :, :]
        for h in range(N_DEV - 1):
            cw_rdma = pltpu.make_async_remote_copy(
                src_ref=cw_ag.at[h], dst_ref=cw_ag.at[h + 1],
                send_sem=cw_ags_sems.at[h], recv_sem=cw_agr_sems.at[h],
                device_id=(right,), device_id_type=pl.DeviceIdType.MESH,
            )
            ccw_rdma = pltpu.make_async_remote_copy(
                src_ref=ccw_ag.at[h], dst_ref=ccw_ag.at[h + 1],
                send_sem=ccw_ags_sems.at[h], recv_sem=ccw_agr_sems.at[h],
                device_id=(left,), device_id_type=pl.DeviceIdType.MESH,
            )
            cw_rdma.start()
            ccw_rdma.start()
            cw_rdma.wait()
            ccw_rdma.wait()
            out_ref[rows(jnp.mod(my_pos - h, N_DEV)), cw_cols] = cw_ag[h + 1, :, :]
            out_ref[rows(jnp.mod(my_pos + h, N_DEV)), ccw_cols] = ccw_ag[h + 1, :, :]

    half = (m_chunk, n_half)
    return pl.pallas_call(
        body,
        out_shape=jax.ShapeDtypeStruct((m_per, n), jnp.float32),
        in_specs=[pl.BlockSpec(memory_space=pltpu.VMEM)],
        out_specs=pl.BlockSpec(memory_space=pltpu.VMEM),
        scratch_shapes=[
            pltpu.VMEM(half, jnp.float32),
            pltpu.VMEM((N_DEV - 1,) + half, jnp.float32),
            pltpu.VMEM((N_DEV,) + half, jnp.float32),
            pltpu.VMEM(half, jnp.float32),
            pltpu.VMEM((N_DEV - 1,) + half, jnp.float32),
            pltpu.VMEM((N_DEV,) + half, jnp.float32),
        ] + [pltpu.SemaphoreType.DMA((N_DEV - 1,)) for _ in range(8)],
        compiler_params=pltpu.CompilerParams(collective_id=0),
    )(t)


# device time: 87830 ns/iter; 1.7768x vs baseline; 1.0113x over previous
import jax
import jax.numpy as jnp
from jax import lax
from jax.experimental import pallas as pl
from jax.experimental.pallas import tpu as pltpu

N_DEV = 4


def kernel(t):
    m_per, n = t.shape
    m_chunk = m_per // N_DEV
    n_half = n // 2

    def body(t_ref, out_ref, cw_rs, ccw_rs,
             cw_s_sems, cw_r_sems, cw_ags_sems, cw_agr_sems,
             ccw_s_sems, ccw_r_sems, ccw_ags_sems, ccw_agr_sems):
        my_pos = lax.axis_index("i")
        left = jnp.mod(my_pos - 1, N_DEV)
        right = jnp.mod(my_pos + 1, N_DEV)

        def rows(c):
            return pl.ds(jnp.mod(c, N_DEV) * m_chunk, m_chunk)

        cw_cols = pl.ds(0, n_half)
        ccw_cols = pl.ds(n_half, n_half)

        barrier_sem = pltpu.get_barrier_semaphore()
        for nbr in [left, right]:
            pl.semaphore_signal(
                barrier_sem, inc=1,
                device_id=(nbr,), device_id_type=pl.DeviceIdType.MESH,
            )
        pl.semaphore_wait(barrier_sem, 2)

        def rs_rdma(h, dir_is_cw):
            if dir_is_cw:
                src = t_ref.at[rows(my_pos), cw_cols] if h == 0 else cw_rs.at[h - 1]
                return pltpu.make_async_remote_copy(
                    src_ref=src, dst_ref=cw_rs.at[h],
                    send_sem=cw_s_sems.at[h], recv_sem=cw_r_sems.at[h],
                    device_id=(right,), device_id_type=pl.DeviceIdType.MESH,
                )
            src = t_ref.at[rows(my_pos), ccw_cols] if h == 0 else ccw_rs.at[h - 1]
            return pltpu.make_async_remote_copy(
                src_ref=src, dst_ref=ccw_rs.at[h],
                send_sem=ccw_s_sems.at[h], recv_sem=ccw_r_sems.at[h],
                device_id=(left,), device_id_type=pl.DeviceIdType.MESH,
            )

        def ag_rdma(h, dir_is_cw):
            if dir_is_cw:
                c_send = my_pos + 1 - h
                return pltpu.make_async_remote_copy(
                    src_ref=out_ref.at[rows(c_send), cw_cols],
                    dst_ref=out_ref.at[rows(c_send), cw_cols],
                    send_sem=cw_ags_sems.at[h], recv_sem=cw_agr_sems.at[h],
                    device_id=(right,), device_id_type=pl.DeviceIdType.MESH,
                )
            c_send = my_pos - 1 + h
            return pltpu.make_async_remote_copy(
                src_ref=out_ref.at[rows(c_send), ccw_cols],
                dst_ref=out_ref.at[rows(c_send), ccw_cols],
                send_sem=ccw_ags_sems.at[h], recv_sem=ccw_agr_sems.at[h],
                device_id=(left,), device_id_type=pl.DeviceIdType.MESH,
            )

        def f_into_out(s, c_own, cols):
            r = jnp.maximum(s, 0.0)
            out_ref[rows(c_own), cols] = jnp.tanh(s) * s * s + r * r * r

        rs_cw = [rs_rdma(h, True) for h in range(N_DEV - 1)]
        rs_ccw = [rs_rdma(h, False) for h in range(N_DEV - 1)]
        ag_cw = [ag_rdma(h, True) for h in range(N_DEV - 1)]
        ag_ccw = [ag_rdma(h, False) for h in range(N_DEV - 1)]

        rs_cw[0].start()
        rs_ccw[0].start()
        for h in range(N_DEV - 1):
            rs_cw[h].wait()
            if h < N_DEV - 2:
                cw_rs[h] = cw_rs[h] + t_ref[rows(my_pos - h - 1), cw_cols]
                rs_cw[h + 1].start()
            else:
                f_into_out(cw_rs[h] + t_ref[rows(my_pos - h - 1), cw_cols],
                           my_pos + 1, cw_cols)
                ag_cw[0].start()
            rs_ccw[h].wait()
            if h < N_DEV - 2:
                ccw_rs[h] = ccw_rs[h] + t_ref[rows(my_pos + h + 1), ccw_cols]
                rs_ccw[h + 1].start()
            else:
                f_into_out(ccw_rs[h] + t_ref[rows(my_pos + h + 1), ccw_cols],
                           my_pos - 1, ccw_cols)
                ag_ccw[0].start()

        for h in range(N_DEV - 1):
            ag_cw[h].wait()
            if h < N_DEV - 2:
                ag_cw[h + 1].start()
            ag_ccw[h].wait()
            if h < N_DEV - 2:
                ag_ccw[h + 1].start()

    half = (m_chunk, n_half)
    return pl.pallas_call(
        body,
        out_shape=jax.ShapeDtypeStruct((m_per, n), jnp.float32),
        in_specs=[pl.BlockSpec(memory_space=pltpu.VMEM)],
        out_specs=pl.BlockSpec(memory_space=pltpu.VMEM),
        scratch_shapes=[
            pltpu.VMEM((N_DEV - 1,) + half, jnp.float32),
            pltpu.VMEM((N_DEV - 1,) + half, jnp.float32),
        ] + [pltpu.SemaphoreType.DMA((N_DEV - 1,)) for _ in range(8)],
        compiler_params=pltpu.CompilerParams(collective_id=0),
    )(t)


# device time: 79771 ns/iter; 1.9564x vs baseline; 1.1010x over previous
import jax
import jax.numpy as jnp
from jax import lax
from jax.experimental import pallas as pl
from jax.experimental.pallas import tpu as pltpu

N_DEV = 4
SEG = 4


def kernel(t):
    m_per, n = t.shape
    m_chunk = m_per // N_DEV
    n_half = n // 2
    n_seg = n_half // SEG

    def body(t_ref, out_ref, cw_rs, ccw_rs,
             cw_s_sems, cw_r_sems, cw_ags_sems, cw_agr_sems,
             ccw_s_sems, ccw_r_sems, ccw_ags_sems, ccw_agr_sems):
        my_pos = lax.axis_index("i")
        left = jnp.mod(my_pos - 1, N_DEV)
        right = jnp.mod(my_pos + 1, N_DEV)

        def rows(c):
            return pl.ds(jnp.mod(c, N_DEV) * m_chunk, m_chunk)

        def glob_cols(dir_is_cw, s):
            base = 0 if dir_is_cw else n_half
            return pl.ds(base + s * n_seg, n_seg)

        def rs_cols(s):
            return pl.ds(s * n_seg, n_seg)

        barrier_sem = pltpu.get_barrier_semaphore()
        for nbr in [left, right]:
            pl.semaphore_signal(
                barrier_sem, inc=1,
                device_id=(nbr,), device_id_type=pl.DeviceIdType.MESH,
            )
        pl.semaphore_wait(barrier_sem, 2)

        def rs_rdma(h, dir_is_cw, s):
            buf, ssem, rsem, tgt = (
                (cw_rs, cw_s_sems, cw_r_sems, right) if dir_is_cw
                else (ccw_rs, ccw_s_sems, ccw_r_sems, left)
            )
            if h == 0:
                src = t_ref.at[rows(my_pos), glob_cols(dir_is_cw, s)]
            else:
                src = buf.at[h - 1, :, rs_cols(s)]
            return pltpu.make_async_remote_copy(
                src_ref=src, dst_ref=buf.at[h, :, rs_cols(s)],
                send_sem=ssem.at[h, s], recv_sem=rsem.at[h, s],
                device_id=(tgt,), device_id_type=pl.DeviceIdType.MESH,
            )

        def ag_rdma(h, dir_is_cw, s):
            if dir_is_cw:
                c_send, ssem, rsem, tgt = my_pos + 1 - h, cw_ags_sems, cw_agr_sems, right
            else:
                c_send, ssem, rsem, tgt = my_pos - 1 + h, ccw_ags_sems, ccw_agr_sems, left
            sl = out_ref.at[rows(c_send), glob_cols(dir_is_cw, s)]
            return pltpu.make_async_remote_copy(
                src_ref=sl, dst_ref=sl,
                send_sem=ssem.at[h, s], recv_sem=rsem.at[h, s],
                device_id=(tgt,), device_id_type=pl.DeviceIdType.MESH,
            )

        rs = {(h, d, s): rs_rdma(h, d, s)
              for h in range(N_DEV - 1) for d in (True, False) for s in range(SEG)}
        ag = {(h, d, s): ag_rdma(h, d, s)
              for h in range(N_DEV - 1) for d in (True, False) for s in range(SEG)}

        for s in range(SEG):
            rs[0, True, s].start()
            rs[0, False, s].start()

        for h in range(N_DEV - 1):
            for s in range(SEG):
                for d in (True, False):
                    rs[h, d, s].wait()
                    buf = cw_rs if d else ccw_rs
                    c_in = my_pos - h - 1 if d else my_pos + h + 1
                    mine = t_ref[rows(c_in), glob_cols(d, s)]
                    if h < N_DEV - 2:
                        buf[h, :, rs_cols(s)] = buf[h, :, rs_cols(s)] + mine
                        rs[h + 1, d, s].start()
                    else:
                        sv = buf[h, :, rs_cols(s)] + mine
                        r = jnp.maximum(sv, 0.0)
                        c_own = my_pos + 1 if d else my_pos - 1
                        out_ref[rows(c_own), glob_cols(d, s)] = (
                            jnp.tanh(sv) * sv * sv + r * r * r
                        )
                        ag[0, d, s].start()

        for h in range(N_DEV - 1):
            for s in range(SEG):
                for d in (True, False):
                    ag[h, d, s].wait()
                    if h < N_DEV - 2:
                        ag[h + 1, d, s].start()

    half = (m_chunk, n_half)
    sem = pltpu.SemaphoreType.DMA((N_DEV - 1, SEG))
    return pl.pallas_call(
        body,
        out_shape=jax.ShapeDtypeStruct((m_per, n), jnp.float32),
        in_specs=[pl.BlockSpec(memory_space=pltpu.VMEM)],
        out_specs=pl.BlockSpec(memory_space=pltpu.VMEM),
        scratch_shapes=[
            pltpu.VMEM((N_DEV - 1,) + half, jnp.float32),
            pltpu.VMEM((N_DEV - 1,) + half, jnp.float32),
        ] + [sem for _ in range(8)],
        compiler_params=pltpu.CompilerParams(collective_id=0),
    )(t)
